# Initial kernel scaffold; baseline (speedup 1.0000x reference)
import jax
import jax.numpy as jnp
from jax.experimental import pallas as pl


def kernel(input, table):
    raise NotImplementedError("write your pallas kernel here")


# speedup vs baseline: 1.4977x; 1.4977x over previous
import functools

import jax
import jax.numpy as jnp
from jax import lax
from jax.experimental import pallas as pl
from jax.experimental.pallas import tpu as pltpu
from jax.experimental.pallas import tpu_sc as plsc

BATCH = 4096
HIST = 200
EMBED = 32
TOTAL = BATCH * HIST

NC = 2
NS = 16
NW = NC * NS
BPW = TOTAL // NW

IDXW = 128
CHUNK = 512
SUB = CHUNK // IDXW
NCHUNK = BPW // CHUNK
NIDX = BPW // IDXW

_mesh = plsc.VectorSubcoreMesh(core_axis_name="c", subcore_axis_name="s")


@functools.partial(
    pl.kernel,
    mesh=_mesh,
    out_type=jax.ShapeDtypeStruct((TOTAL, EMBED), jnp.float32),
    compiler_params=pltpu.CompilerParams(use_tc_tiling_on_sc=False),
    scratch_types=[
        pltpu.VMEM((NIDX, IDXW), jnp.int32),
        pltpu.VMEM((CHUNK, EMBED), jnp.float32),
        pltpu.VMEM((CHUNK, EMBED), jnp.float32),
        pltpu.SemaphoreType.DMA,
        pltpu.SemaphoreType.DMA,
        pltpu.SemaphoreType.DMA,
        pltpu.SemaphoreType.DMA,
    ],
)
def _gather_kernel(table_hbm, idx_hbm, out_hbm, idx_v, rows0, rows1,
                   gsem0, gsem1, wsem0, wsem1):
    wid = lax.axis_index("s") * NC + lax.axis_index("c")
    base = wid * BPW

    pltpu.sync_copy(idx_hbm.at[wid], idx_v)

    rows = (rows0, rows1)
    gsems = (gsem0, gsem1)
    wsems = (wsem0, wsem1)

    def start_gather(chunk, b):
        for s in range(SUB):
            pltpu.async_copy(
                table_hbm.at[idx_v.at[chunk * SUB + s]],
                rows[b].at[pl.ds(s * IDXW, IDXW)],
                gsems[b],
            )

    def wait_gather(chunk, b):
        for s in range(SUB):
            pltpu.make_async_copy(
                table_hbm.at[idx_v.at[chunk * SUB + s]],
                rows[b].at[pl.ds(s * IDXW, IDXW)],
                gsems[b],
            ).wait()

    def start_write(chunk, b):
        pltpu.async_copy(
            rows[b], out_hbm.at[pl.ds(base + chunk * CHUNK, CHUNK)], wsems[b])

    def wait_write(chunk, b):
        pltpu.make_async_copy(
            rows[b], out_hbm.at[pl.ds(base + chunk * CHUNK, CHUNK)], wsems[b]
        ).wait()

    start_gather(0, 0)

    @pl.loop(0, NCHUNK, step=2)
    def _body(j):
        for b in range(2):
            jj = j + b
            nb = (b + 1) % 2

            @pl.when(jj + 1 < NCHUNK)
            def _():
                @pl.when(jj >= 1)
                def _():
                    wait_write(jj - 1, nb)

                start_gather(jj + 1, nb)

            wait_gather(jj, b)
            start_write(jj, b)

    wait_write(NCHUNK - 2, 0)
    wait_write(NCHUNK - 1, 1)


def kernel(input, table):
    idx = input.reshape(-1).astype(jnp.int32).reshape(NW, NIDX, IDXW)
    out = _gather_kernel(table, idx)
    return out.reshape(BATCH, HIST, EMBED)
